# baseline (speedup 1.0000x reference)
import functools

import jax
import jax.numpy as jnp
from jax import lax
from jax.experimental import pallas as pl
from jax.experimental.pallas import tpu as pltpu
from jax.experimental.pallas import tpu_sc as plsc

_LANES = 16


def _make_tc_transpose(V, D, TB=32768):
    H = TB // 2
    n_blk = (V + TB - 1) // TB
    Vp = n_blk * TB

    def body(x_ref, o_ref):
        xt = x_ref[...].T
        o_ref[...] = jnp.concatenate(
            [xt[:H], xt[H:]], axis=1).astype(jnp.bfloat16)

    return pl.pallas_call(
        body,
        grid=(n_blk,),
        in_specs=[pl.BlockSpec((D, TB), lambda j: (0, j))],
        out_specs=pl.BlockSpec((H, 2 * D), lambda j: (j, 0)),
        out_shape=jax.ShapeDtypeStruct((Vp // 2, 2 * D), jnp.bfloat16),
        compiler_params=pltpu.CompilerParams(
            dimension_semantics=("parallel",),
            vmem_limit_bytes=100 * 1024 * 1024),
    ), Vp


def _row64_map(v, TB=32768):
    H = TB // 2
    return (v // TB) * TB + 2 * (v % H) + (v % TB) // H


def _make_transpose_kernel(V, D):
    info = plsc.get_sparse_core_info()
    NC, NS = info.num_cores, info.num_subcores
    NW = NC * NS
    TB = 128
    n_full = V // TB
    rem = V - n_full * TB
    per_w = n_full // NW
    n_extra = n_full - per_w * NW
    n_d = D // _LANES

    mesh = plsc.VectorSubcoreMesh(core_axis_name="c", subcore_axis_name="s")

    @functools.partial(
        pl.kernel,
        mesh=mesh,
        compiler_params=pltpu.CompilerParams(
            use_tc_tiling_on_sc=True, needs_layout_passes=False),
        out_type=jax.ShapeDtypeStruct((V // 2, 2 * D), jnp.float32),
        scratch_types=[
            pltpu.VMEM((2, D, TB), jnp.float32),
            pltpu.VMEM((2, TB // 2, 2 * D), jnp.float32),
            pltpu.SemaphoreType.DMA,
            pltpu.SemaphoreType.DMA,
            pltpu.SemaphoreType.DMA,
            pltpu.SemaphoreType.DMA,
        ],
    )
    def k(tt_hbm, patch_hbm, out_hbm, in_v, tr_v, gi0, gi1, go0, go1):
        wid = lax.axis_index("s") * NC + lax.axis_index("c")
        gsems = (gi0, gi1)
        osems = (go0, go1)

        dvecs = [jnp.arange(_LANES, dtype=jnp.int32) + c * _LANES
                 for c in range(n_d)]

        def issue_in(blk, b):
            pltpu.async_copy(
                tt_hbm.at[:, pl.ds(blk * TB, TB)], in_v.at[b], gsems[b])

        def drain_in(b):
            pltpu.make_async_copy(
                tt_hbm.at[:, pl.ds(0, TB)], in_v.at[b], gsems[b]).wait()

        def drain_out(b):
            pltpu.make_async_copy(
                tt_hbm.at[:, pl.ds(0, TB)], tr_v.at[b], osems[b]).wait()

        def transpose_into(b, ntok):
            def pairrow(p, _):
                for half in range(2):
                    rv = jnp.full((_LANES,), 2 * p + half, jnp.int32)
                    for c in range(n_d):
                        e = plsc.load_gather(in_v.at[b], [dvecs[c], rv])
                        tr_v[b, p, pl.ds(half * D + c * _LANES, _LANES)] = e
                return None
            lax.fori_loop(0, ntok // 2, pairrow, None, unroll=2)

        def flush(blk, b):
            pltpu.async_copy(
                tr_v.at[b], out_hbm.at[pl.ds(blk * (TB // 2), TB // 2)],
                osems[b])

        def blk_of(i):
            return wid * per_w + i

        issue_in(blk_of(0), 0)

        def pair_body(j, _):
            i0 = 2 * j
            issue_in(blk_of(i0 + 1), 1)
            drain_in(0)

            @pl.when(j > 0)
            def _():
                drain_out(0)

            transpose_into(0, TB)
            flush(blk_of(i0), 0)

            @pl.when(i0 + 2 < per_w)
            def _():
                issue_in(blk_of(i0 + 2), 0)

            drain_in(1)

            @pl.when(j > 0)
            def _():
                drain_out(1)

            transpose_into(1, TB)
            flush(blk_of(i0 + 1), 1)
            return None

        lax.fori_loop(0, per_w // 2, pair_body, None)
        drain_out(0)
        drain_out(1)

        @pl.when(wid < n_extra)
        def _():
            blk = n_full - n_extra + wid
            pltpu.sync_copy(tt_hbm.at[:, pl.ds(blk * TB, TB)], in_v.at[0])
            transpose_into(0, TB)
            pltpu.sync_copy(
                tr_v.at[0], out_hbm.at[pl.ds(blk * (TB // 2), TB // 2)])

        if rem:
            @pl.when(wid == n_extra)
            def _():
                pltpu.sync_copy(patch_hbm, tr_v.at[0, pl.ds(0, rem // 2)])
                pltpu.sync_copy(
                    tr_v.at[0, pl.ds(0, rem // 2)],
                    out_hbm.at[pl.ds(n_full * (TB // 2), rem // 2)])

    return k


def _make_gather_kernel(B, L, D, V):
    info = plsc.get_sparse_core_info()
    NC, NS = info.num_cores, info.num_subcores
    NW = NC * NS
    assert B % NW == 0
    b_per_w = B // NW
    n_d = D // _LANES
    half = L // 2
    OUT = 3 * D

    mesh = plsc.VectorSubcoreMesh(core_axis_name="c", subcore_axis_name="s")

    @functools.partial(
        pl.kernel,
        mesh=mesh,
        compiler_params=pltpu.CompilerParams(
            use_tc_tiling_on_sc=False, needs_layout_passes=False),
        out_type=jax.ShapeDtypeStruct((B, OUT), jnp.float32),
        scratch_types=[
            pltpu.VMEM((b_per_w, 2, half), jnp.int32),
            pltpu.VMEM((2, L, D), jnp.bfloat16),
            pltpu.VMEM((b_per_w, OUT), jnp.float32),
            pltpu.SemaphoreType.DMA,
            pltpu.SemaphoreType.DMA,
        ],
    )
    def k(tok_hbm, table_hbm, out_hbm, idx_v, rows_v, out_v, sem0, sem1):
        wid = lax.axis_index("s") * NC + lax.axis_index("c")
        base = wid * b_per_w

        pltpu.sync_copy(tok_hbm.at[pl.ds(base, b_per_w)], idx_v)

        inv1 = 1.0 / L
        inv2 = 1.0 / (L - 1)
        inv3 = 1.0 / (L - 2)

        def issue(i, b, sem):
            pltpu.async_copy(
                table_hbm.at[idx_v.at[i, 0]], rows_v.at[b, pl.ds(0, half)], sem)
            pltpu.async_copy(
                table_hbm.at[idx_v.at[i, 1]], rows_v.at[b, pl.ds(half, half)], sem)

        def drain(b, sem):
            for s in range(2):
                pltpu.make_async_copy(
                    table_hbm.at[idx_v.at[0, 0]],
                    rows_v.at[b, pl.ds(s * half, half)], sem).wait()

        n_g = D // 32
        colvecs = [
            [jnp.arange(_LANES, dtype=jnp.int32) * 2 + (32 * g + par)
             for par in range(2)]
            for g in range(n_g)
        ]

        def compute(i, b):
            def step(l, carry):
                new = []
                for g in range(n_g):
                    x = rows_v[b, l, pl.ds(g * 32, 32)]
                    es = plsc.unpack(
                        x, format=plsc.PackFormat.INTERLEAVED,
                        preferred_element_type=jnp.float32)
                    for par in range(2):
                        j5 = 5 * (2 * g + par)
                        e_prev, pair_prev, a1, a2, a3 = carry[j5:j5 + 5]
                        e = es[par]
                        pair = e_prev * e
                        trip = pair_prev * e
                        new.extend((e, pair, a1 + e, a2 + pair, a3 + trip))
                return tuple(new)

            zeros = jnp.zeros((_LANES,), jnp.float32)
            carry = tuple(zeros for _ in range(5 * 2 * n_g))
            carry = lax.fori_loop(0, L, step, carry, unroll=8)
            iv = jnp.full((_LANES,), i, jnp.int32)
            for g in range(n_g):
                for par in range(2):
                    j5 = 5 * (2 * g + par)
                    _, _, a1, a2, a3 = carry[j5:j5 + 5]
                    cv = colvecs[g][par]
                    plsc.store_scatter(out_v, [iv, cv], a1 * inv1)
                    plsc.store_scatter(out_v, [iv, cv + D], a2 * inv2)
                    plsc.store_scatter(out_v, [iv, cv + 2 * D], a3 * inv3)

        issue(0, 0, sem0)

        def pair_body(j, _):
            i0 = 2 * j
            issue(i0 + 1, 1, sem1)
            drain(0, sem0)
            compute(i0, 0)

            @pl.when(i0 + 2 < b_per_w)
            def _():
                issue(i0 + 2, 0, sem0)

            drain(1, sem1)
            compute(i0 + 1, 1)
            return None

        lax.fori_loop(0, b_per_w // 2, pair_body, None)

        pltpu.sync_copy(out_v, out_hbm.at[pl.ds(base, b_per_w)])

    return k


def kernel(token_ids, table):
    B, L = token_ids.shape
    V, D = table.shape
    tok_m = _row64_map(token_ids.astype(jnp.int32))
    tok3 = tok_m.reshape(B, 2, L // 2)
    tk, Vp = _make_tc_transpose(V, D)
    table_rm = tk(table.T).reshape(Vp, D)
    gk = _make_gather_kernel(B, L, D, Vp)
    return gk(tok3, table_rm)


# speedup vs baseline: 1.7768x; 1.7768x over previous
import functools

import jax
import jax.numpy as jnp
from jax import lax
from jax.experimental import pallas as pl
from jax.experimental.pallas import tpu as pltpu
from jax.experimental.pallas import tpu_sc as plsc

_LANES = 16


def _make_tc_transpose(V, D, TB=32768):
    H = TB // 2
    n_blk = (V + TB - 1) // TB
    Vp = n_blk * TB

    def body(x_ref, o_ref):
        xt = x_ref[...].T
        o_ref[...] = jnp.concatenate([xt[:H], xt[H:]], axis=1)

    return pl.pallas_call(
        body,
        grid=(n_blk,),
        in_specs=[pl.BlockSpec((D, TB), lambda j: (0, j))],
        out_specs=pl.BlockSpec((H, 2 * D), lambda j: (j, 0)),
        out_shape=jax.ShapeDtypeStruct((Vp // 2, 2 * D), jnp.float32),
        compiler_params=pltpu.CompilerParams(
            dimension_semantics=("parallel",),
            vmem_limit_bytes=100 * 1024 * 1024),
    ), Vp


def _row64_map(v, TB=32768):
    H = TB // 2
    return (v // TB) * TB + 2 * (v % H) + (v % TB) // H


def _make_transpose_kernel(V, D):
    info = plsc.get_sparse_core_info()
    NC, NS = info.num_cores, info.num_subcores
    NW = NC * NS
    TB = 128
    n_full = V // TB
    rem = V - n_full * TB
    per_w = n_full // NW
    n_extra = n_full - per_w * NW
    n_d = D // _LANES

    mesh = plsc.VectorSubcoreMesh(core_axis_name="c", subcore_axis_name="s")

    @functools.partial(
        pl.kernel,
        mesh=mesh,
        compiler_params=pltpu.CompilerParams(
            use_tc_tiling_on_sc=True, needs_layout_passes=False),
        out_type=jax.ShapeDtypeStruct((V // 2, 2 * D), jnp.float32),
        scratch_types=[
            pltpu.VMEM((2, D, TB), jnp.float32),
            pltpu.VMEM((2, TB // 2, 2 * D), jnp.float32),
            pltpu.SemaphoreType.DMA,
            pltpu.SemaphoreType.DMA,
            pltpu.SemaphoreType.DMA,
            pltpu.SemaphoreType.DMA,
        ],
    )
    def k(tt_hbm, patch_hbm, out_hbm, in_v, tr_v, gi0, gi1, go0, go1):
        wid = lax.axis_index("s") * NC + lax.axis_index("c")
        gsems = (gi0, gi1)
        osems = (go0, go1)

        dvecs = [jnp.arange(_LANES, dtype=jnp.int32) + c * _LANES
                 for c in range(n_d)]

        def issue_in(blk, b):
            pltpu.async_copy(
                tt_hbm.at[:, pl.ds(blk * TB, TB)], in_v.at[b], gsems[b])

        def drain_in(b):
            pltpu.make_async_copy(
                tt_hbm.at[:, pl.ds(0, TB)], in_v.at[b], gsems[b]).wait()

        def drain_out(b):
            pltpu.make_async_copy(
                tt_hbm.at[:, pl.ds(0, TB)], tr_v.at[b], osems[b]).wait()

        def transpose_into(b, ntok):
            def pairrow(p, _):
                for half in range(2):
                    rv = jnp.full((_LANES,), 2 * p + half, jnp.int32)
                    for c in range(n_d):
                        e = plsc.load_gather(in_v.at[b], [dvecs[c], rv])
                        tr_v[b, p, pl.ds(half * D + c * _LANES, _LANES)] = e
                return None
            lax.fori_loop(0, ntok // 2, pairrow, None, unroll=2)

        def flush(blk, b):
            pltpu.async_copy(
                tr_v.at[b], out_hbm.at[pl.ds(blk * (TB // 2), TB // 2)],
                osems[b])

        def blk_of(i):
            return wid * per_w + i

        issue_in(blk_of(0), 0)

        def pair_body(j, _):
            i0 = 2 * j
            issue_in(blk_of(i0 + 1), 1)
            drain_in(0)

            @pl.when(j > 0)
            def _():
                drain_out(0)

            transpose_into(0, TB)
            flush(blk_of(i0), 0)

            @pl.when(i0 + 2 < per_w)
            def _():
                issue_in(blk_of(i0 + 2), 0)

            drain_in(1)

            @pl.when(j > 0)
            def _():
                drain_out(1)

            transpose_into(1, TB)
            flush(blk_of(i0 + 1), 1)
            return None

        lax.fori_loop(0, per_w // 2, pair_body, None)
        drain_out(0)
        drain_out(1)

        @pl.when(wid < n_extra)
        def _():
            blk = n_full - n_extra + wid
            pltpu.sync_copy(tt_hbm.at[:, pl.ds(blk * TB, TB)], in_v.at[0])
            transpose_into(0, TB)
            pltpu.sync_copy(
                tr_v.at[0], out_hbm.at[pl.ds(blk * (TB // 2), TB // 2)])

        if rem:
            @pl.when(wid == n_extra)
            def _():
                pltpu.sync_copy(patch_hbm, tr_v.at[0, pl.ds(0, rem // 2)])
                pltpu.sync_copy(
                    tr_v.at[0, pl.ds(0, rem // 2)],
                    out_hbm.at[pl.ds(n_full * (TB // 2), rem // 2)])

    return k


def _make_gather_kernel(B, L, D, V):
    info = plsc.get_sparse_core_info()
    NC, NS = info.num_cores, info.num_subcores
    NW = NC * NS
    assert B % NW == 0
    b_per_w = B // NW
    n_d = D // _LANES
    half = L // 2
    OUT = 3 * D

    mesh = plsc.VectorSubcoreMesh(core_axis_name="c", subcore_axis_name="s")

    @functools.partial(
        pl.kernel,
        mesh=mesh,
        compiler_params=pltpu.CompilerParams(use_tc_tiling_on_sc=False),
        out_type=jax.ShapeDtypeStruct((B, OUT), jnp.float32),
        scratch_types=[
            pltpu.VMEM((b_per_w, 2, half), jnp.int32),
            pltpu.VMEM((2, L, D), jnp.float32),
            pltpu.VMEM((b_per_w, OUT), jnp.float32),
            pltpu.SemaphoreType.DMA,
            pltpu.SemaphoreType.DMA,
        ],
    )
    def k(tok_hbm, table_hbm, out_hbm, idx_v, rows_v, out_v, sem0, sem1):
        wid = lax.axis_index("s") * NC + lax.axis_index("c")
        base = wid * b_per_w

        pltpu.sync_copy(tok_hbm.at[pl.ds(base, b_per_w)], idx_v)

        inv1 = 1.0 / L
        inv2 = 1.0 / (L - 1)
        inv3 = 1.0 / (L - 2)

        def issue(i, b, sem):
            pltpu.async_copy(
                table_hbm.at[idx_v.at[i, 0]], rows_v.at[b, pl.ds(0, half)], sem)
            pltpu.async_copy(
                table_hbm.at[idx_v.at[i, 1]], rows_v.at[b, pl.ds(half, half)], sem)

        def drain(b, sem):
            for s in range(2):
                pltpu.make_async_copy(
                    table_hbm.at[idx_v.at[0, 0]],
                    rows_v.at[b, pl.ds(s * half, half)], sem).wait()

        def compute(i, b):
            def step(l, carry):
                new = []
                for c in range(n_d):
                    e_prev, pair_prev, a1, a2, a3 = carry[5 * c:5 * c + 5]
                    e = rows_v[b, l, pl.ds(c * _LANES, _LANES)]
                    pair = e_prev * e
                    trip = pair_prev * e
                    new.extend((e, pair, a1 + e, a2 + pair, a3 + trip))
                return tuple(new)

            zeros = jnp.zeros((_LANES,), jnp.float32)
            carry = tuple(zeros for _ in range(5 * n_d))
            carry = lax.fori_loop(0, L, step, carry, unroll=8)
            for c in range(n_d):
                _, _, a1, a2, a3 = carry[5 * c:5 * c + 5]
                out_v[i, pl.ds(c * _LANES, _LANES)] = a1 * inv1
                out_v[i, pl.ds(D + c * _LANES, _LANES)] = a2 * inv2
                out_v[i, pl.ds(2 * D + c * _LANES, _LANES)] = a3 * inv3

        issue(0, 0, sem0)

        def pair_body(j, _):
            i0 = 2 * j
            issue(i0 + 1, 1, sem1)
            drain(0, sem0)
            compute(i0, 0)

            @pl.when(i0 + 2 < b_per_w)
            def _():
                issue(i0 + 2, 0, sem0)

            drain(1, sem1)
            compute(i0 + 1, 1)
            return None

        lax.fori_loop(0, b_per_w // 2, pair_body, None)

        pltpu.sync_copy(out_v, out_hbm.at[pl.ds(base, b_per_w)])

    return k


def kernel(token_ids, table):
    B, L = token_ids.shape
    V, D = table.shape
    tok_m = _row64_map(token_ids.astype(jnp.int32))
    tok3 = tok_m.reshape(B, 2, L // 2)
    tk, Vp = _make_tc_transpose(V, D)
    table_rm = tk(table.T).reshape(Vp, D)
    gk = _make_gather_kernel(B, L, D, Vp)
    return gk(tok3, table_rm)


# speedup vs baseline: 1.7779x; 1.0006x over previous
import functools

import jax
import jax.numpy as jnp
from jax import lax
from jax.experimental import pallas as pl
from jax.experimental.pallas import tpu as pltpu
from jax.experimental.pallas import tpu_sc as plsc

_LANES = 16


def _make_tc_transpose(V, D, TB=32768):
    H = TB // 2
    n_blk = (V + TB - 1) // TB
    Vp = n_blk * TB

    def body(x_ref, o_ref):
        xt = x_ref[...].T
        o_ref[...] = jnp.concatenate([xt[:H], xt[H:]], axis=1)

    return pl.pallas_call(
        body,
        grid=(n_blk,),
        in_specs=[pl.BlockSpec((D, TB), lambda j: (0, j))],
        out_specs=pl.BlockSpec((H, 2 * D), lambda j: (j, 0)),
        out_shape=jax.ShapeDtypeStruct((Vp // 2, 2 * D), jnp.float32),
        compiler_params=pltpu.CompilerParams(
            dimension_semantics=("parallel",),
            vmem_limit_bytes=100 * 1024 * 1024),
    ), Vp


def _row64_map(v, TB=32768):
    H = TB // 2
    return (v // TB) * TB + 2 * (v % H) + (v % TB) // H


def _make_gather_kernel(B, L, D, V):
    info = plsc.get_sparse_core_info()
    NC, NS = info.num_cores, info.num_subcores
    NW = NC * NS
    assert B % NW == 0
    b_per_w = B // NW
    n_d = D // _LANES
    half = L // 2
    OUT = 3 * D

    mesh = plsc.VectorSubcoreMesh(core_axis_name="c", subcore_axis_name="s")

    @functools.partial(
        pl.kernel,
        mesh=mesh,
        compiler_params=pltpu.CompilerParams(use_tc_tiling_on_sc=False),
        out_type=jax.ShapeDtypeStruct((B, OUT), jnp.float32),
        scratch_types=[
            pltpu.VMEM((b_per_w, 2, half), jnp.int32),
            pltpu.VMEM((2, L, D), jnp.float32),
            pltpu.VMEM((b_per_w, OUT), jnp.float32),
            pltpu.SemaphoreType.DMA,
            pltpu.SemaphoreType.DMA,
        ],
    )
    def k(tok_hbm, table_hbm, out_hbm, idx_v, rows_v, out_v, sem0, sem1):
        wid = lax.axis_index("s") * NC + lax.axis_index("c")
        base = wid * b_per_w

        pltpu.sync_copy(tok_hbm.at[pl.ds(base, b_per_w)], idx_v)

        inv1 = 1.0 / L
        inv2 = 1.0 / (L - 1)
        inv3 = 1.0 / (L - 2)

        def issue(i, b, sem):
            pltpu.async_copy(
                table_hbm.at[idx_v.at[i, 0]], rows_v.at[b, pl.ds(0, half)], sem)
            pltpu.async_copy(
                table_hbm.at[idx_v.at[i, 1]], rows_v.at[b, pl.ds(half, half)], sem)

        def drain(b, sem):
            for s in range(2):
                pltpu.make_async_copy(
                    table_hbm.at[idx_v.at[0, 0]],
                    rows_v.at[b, pl.ds(s * half, half)], sem).wait()

        def compute(i, b):
            def step(l, carry):
                new = []
                for c in range(n_d):
                    e_prev, pair_prev, a1, a2, a3 = carry[5 * c:5 * c + 5]
                    e = rows_v[b, l, pl.ds(c * _LANES, _LANES)]
                    pair = e_prev * e
                    trip = pair_prev * e
                    new.extend((e, pair, a1 + e, a2 + pair, a3 + trip))
                return tuple(new)

            zeros = jnp.zeros((_LANES,), jnp.float32)
            carry = tuple(zeros for _ in range(5 * n_d))
            carry = lax.fori_loop(0, L, step, carry, unroll=8)
            for c in range(n_d):
                _, _, a1, a2, a3 = carry[5 * c:5 * c + 5]
                out_v[i, pl.ds(c * _LANES, _LANES)] = a1 * inv1
                out_v[i, pl.ds(D + c * _LANES, _LANES)] = a2 * inv2
                out_v[i, pl.ds(2 * D + c * _LANES, _LANES)] = a3 * inv3

        issue(0, 0, sem0)

        def pair_body(j, _):
            i0 = 2 * j
            issue(i0 + 1, 1, sem1)
            drain(0, sem0)
            compute(i0, 0)

            @pl.when(i0 + 2 < b_per_w)
            def _():
                issue(i0 + 2, 0, sem0)

            drain(1, sem1)
            compute(i0 + 1, 1)
            return None

        lax.fori_loop(0, b_per_w // 2, pair_body, None)

        pltpu.sync_copy(out_v, out_hbm.at[pl.ds(base, b_per_w)])

    return k


def kernel(token_ids, table):
    B, L = token_ids.shape
    V, D = table.shape
    tok_m = _row64_map(token_ids.astype(jnp.int32))
    tok3 = tok_m.reshape(B, 2, L // 2)
    tk, Vp = _make_tc_transpose(V, D)
    table_rm = tk(table.T).reshape(Vp, D)
    gk = _make_gather_kernel(B, L, D, Vp)
    return gk(tok3, table_rm)
